# baseline (speedup 1.0000x reference)
import jax
import jax.numpy as jnp
from jax.experimental import pallas as pl
from jax.experimental.pallas import tpu as pltpu

DIM = 512
CODEBOOK_SIZE = 1024
CODEBOOK_DIM = 64
NUM_CODEBOOKS = 2

ROW_TILE = 512


def _rpq_kernel(x_ref, p_ref, emb_ref, i0_ref, i1_ref):
    x = x_ref[...]
    p = p_ref[...]

    mu = jnp.mean(x, axis=-1, keepdims=True)
    xc = x - mu
    var = jnp.mean(xc * xc, axis=-1, keepdims=True)
    xn = xc / jnp.sqrt(var + 1e-5)

    proj = jnp.dot(xn, p, preferred_element_type=jnp.float32)

    for h, out_ref in ((0, i0_ref), (1, i1_ref)):
        cb = emb_ref[h]
        cbn = cb / jnp.clip(
            jnp.sqrt(jnp.sum(cb * cb, axis=-1, keepdims=True)), 1e-12, None)
        ph = proj[:, h * CODEBOOK_DIM:(h + 1) * CODEBOOK_DIM]
        phn = ph / jnp.clip(
            jnp.sqrt(jnp.sum(ph * ph, axis=-1, keepdims=True)), 1e-12, None)
        scores = jnp.dot(phn, cbn.T, preferred_element_type=jnp.float32)
        out_ref[...] = jnp.argmax(scores, axis=-1).astype(jnp.int32)


def kernel(x, rand_projs, embed):
    b, n, d = x.shape
    m = b * n
    xf = x.reshape(m, d)
    p = rand_projs.transpose(1, 0, 2).reshape(d, NUM_CODEBOOKS * CODEBOOK_DIM)

    grid = (m // ROW_TILE,)
    out_shape = [jax.ShapeDtypeStruct((m,), jnp.int32) for _ in range(2)]
    i0, i1 = pl.pallas_call(
        _rpq_kernel,
        grid=grid,
        in_specs=[
            pl.BlockSpec((ROW_TILE, d), lambda i: (i, 0)),
            pl.BlockSpec((d, NUM_CODEBOOKS * CODEBOOK_DIM), lambda i: (0, 0)),
            pl.BlockSpec((NUM_CODEBOOKS, CODEBOOK_SIZE, CODEBOOK_DIM),
                         lambda i: (0, 0, 0)),
        ],
        out_specs=[pl.BlockSpec((ROW_TILE,), lambda i: (i,)) for _ in range(2)],
        out_shape=out_shape,
        compiler_params=pltpu.CompilerParams(
            dimension_semantics=("parallel",)),
    )(xf, p, embed)
    return jnp.stack([i0, i1], axis=-1).reshape(b, n, NUM_CODEBOOKS)


# speedup vs baseline: 1.0365x; 1.0365x over previous
import jax
import jax.numpy as jnp
from jax.experimental import pallas as pl
from jax.experimental.pallas import tpu as pltpu

DIM = 512
CODEBOOK_SIZE = 1024
CODEBOOK_DIM = 64
NUM_CODEBOOKS = 2

ROW_TILE = 512


def _rpq_kernel(x_ref, p_ref, emb_ref, out_ref):
    x = x_ref[...]
    p = p_ref[...]

    mu = jnp.mean(x, axis=-1, keepdims=True)
    xc = x - mu
    var = jnp.mean(xc * xc, axis=-1, keepdims=True)
    xn = xc / jnp.sqrt(var + 1e-5)

    proj = jnp.dot(xn, p, preferred_element_type=jnp.float32)

    idxs = []
    for h in range(NUM_CODEBOOKS):
        cb = emb_ref[h]
        cbn = cb / jnp.clip(
            jnp.sqrt(jnp.sum(cb * cb, axis=-1, keepdims=True)), 1e-12, None)
        ph = proj[:, h * CODEBOOK_DIM:(h + 1) * CODEBOOK_DIM]
        phn = ph / jnp.clip(
            jnp.sqrt(jnp.sum(ph * ph, axis=-1, keepdims=True)), 1e-12, None)
        scores = jnp.dot(phn, cbn.T, preferred_element_type=jnp.float32)
        idxs.append(jnp.argmax(scores, axis=-1).astype(jnp.int32))
    out_ref[...] = jnp.stack(idxs, axis=-1)


def kernel(x, rand_projs, embed):
    b, n, d = x.shape
    m = b * n
    xf = x.reshape(m, d)
    p = rand_projs.transpose(1, 0, 2).reshape(d, NUM_CODEBOOKS * CODEBOOK_DIM)

    grid = (m // ROW_TILE,)
    out_shape = jax.ShapeDtypeStruct((m, NUM_CODEBOOKS), jnp.int32)
    out = pl.pallas_call(
        _rpq_kernel,
        grid=grid,
        in_specs=[
            pl.BlockSpec((ROW_TILE, d), lambda i: (i, 0)),
            pl.BlockSpec((d, NUM_CODEBOOKS * CODEBOOK_DIM), lambda i: (0, 0)),
            pl.BlockSpec((NUM_CODEBOOKS, CODEBOOK_SIZE, CODEBOOK_DIM),
                         lambda i: (0, 0, 0)),
        ],
        out_specs=pl.BlockSpec((ROW_TILE, NUM_CODEBOOKS), lambda i: (i, 0)),
        out_shape=out_shape,
        compiler_params=pltpu.CompilerParams(
            dimension_semantics=("parallel",)),
    )(xf, p, embed)
    return out.reshape(b, n, NUM_CODEBOOKS)
